# Initial kernel scaffold; baseline (speedup 1.0000x reference)
import jax
import jax.numpy as jnp
from jax.experimental import pallas as pl


def kernel(user_emb, movie_emb, edge_index):
    raise NotImplementedError("write your pallas kernel here")


# speedup vs baseline: 12.3320x; 12.3320x over previous
import functools

import jax
import jax.numpy as jnp
from jax import lax
from jax.experimental import pallas as pl
from jax.experimental.pallas import tpu as pltpu
from jax.experimental.pallas import tpu_sc as plsc

N_USERS = 25000
N_MOVIES = 25000
N_NODES = 50000
D = 64
HALF = 25000
PAD = 25088
SLICE = PAD // 16
XC = SLICE // 14
W = 128
E = 800000
ESL = E // 16
CAP = 50048
CHUNK = 2000
NTRASH = PAD - HALF

@functools.lru_cache(maxsize=None)
def _mesh():
    return plsc.VectorSubcoreMesh(core_axis_name="c", subcore_axis_name="s")


def _zero_rows(buf, nrows):
    zv = jnp.zeros((16,), jnp.float32)

    def body(r, _):
        for q in range(D // 16):
            buf[r, pl.ds(q * 16, 16)] = zv
        return 0

    lax.fori_loop(0, nrows, body, 0)


def _route_body(src_hbm, dst_hbm, xpad_hbm,
                esrc_hbm, edst_hbm, ecnt_hbm, dis_hbm, z0_hbm,
                srcb, dstb, lsrc, ldst, idxb, onesb, degv, cntb, xb, sdeg, sem):
    c = lax.axis_index("c")
    s = lax.axis_index("s")
    w = c * 16 + s
    base = c * HALF
    lanes = lax.iota(jnp.int32, 16)
    myrow = s * SLICE

    def zbody(k, _):
        degv[pl.ds(k * 16, 16)] = jnp.zeros((16,), jnp.float32)
        return 0
    lax.fori_loop(0, SLICE // 16, zbody, 0)
    pltpu.sync_copy(degv, sdeg.at[pl.ds(myrow, SLICE)])
    plsc.subcore_barrier()

    e0 = s * ESL

    def chunk_body(i, cnt):
        off = e0 + i * CHUNK
        pltpu.sync_copy(src_hbm.at[pl.ds(off, CHUNK)], srcb)
        pltpu.sync_copy(dst_hbm.at[pl.ds(off, CHUNK)], dstb)

        def vec_body(k, cnt):
            d = dstb[pl.ds(k * 16, 16)]
            sv = srcb[pl.ds(k * 16, 16)]
            inh = (d >= base) & (d < base + HALF)
            dl = d - base
            sp = sv + (PAD - HALF) * (sv >= HALF).astype(jnp.int32)
            cs = plsc.cumsum(inh.astype(jnp.int32))
            pos = cnt + cs - 1
            plsc.store_scatter(lsrc, [pos], sp, mask=inh)
            plsc.store_scatter(ldst, [pos], dl, mask=inh)
            return cnt + cs[15]

        return lax.fori_loop(0, CHUNK // 16, vec_body, cnt)

    cnt = lax.fori_loop(0, ESL // CHUNK, chunk_body, 0)
    cntp = ((cnt + W - 1) // W) * W

    for j in range(W // 16):
        lsrc[pl.ds(cnt + j * 16, 16)] = s * 97 + j * 16 + lanes
        ldst[pl.ds(cnt + j * 16, 16)] = HALF + ((j * 16 + lanes) % NTRASH)

    pltpu.sync_copy(lsrc, esrc_hbm.at[pl.ds(w * CAP, CAP)])
    pltpu.sync_copy(ldst, edst_hbm.at[pl.ds(w * CAP, CAP)])
    cntb[pl.ds(0, 16)] = jnp.full((16,), cntp, jnp.int32)
    pltpu.sync_copy(cntb, ecnt_hbm.at[pl.ds(w * 16, 16)])

    def obody(j, _):
        onesb[pl.ds(j * 16, 16)] = jnp.ones((16,), jnp.float32)
        return 0
    lax.fori_loop(0, W // 16, obody, 0)
    nwin = cntp // W

    def deg_body(g, _):
        def cp(j, _):
            idxb[pl.ds(j * 16, 16)] = ldst[pl.ds(g * W + j * 16, 16)]
            return 0
        lax.fori_loop(0, W // 16, cp, 0)
        pltpu.sync_copy(onesb, sdeg.at[idxb], add=True)
        return 0
    lax.fori_loop(0, nwin, deg_body, 0)
    plsc.subcore_barrier()

    pltpu.sync_copy(sdeg.at[pl.ds(myrow, SLICE)], degv)

    def nbody(k, _):
        dgv = degv[pl.ds(k * 16, 16)]
        bi = lax.bitcast_convert_type(dgv, jnp.int32)
        y = lax.bitcast_convert_type(
            jnp.int32(0x5F3759DF) - lax.shift_right_logical(bi, 1), jnp.float32)
        for _ in range(3):
            y = y * (1.5 - 0.5 * dgv * y * y)
        degv[pl.ds(k * 16, 16)] = jnp.where(dgv >= 1.0, y, 0.0)
        return 0
    lax.fori_loop(0, SLICE // 16, nbody, 0)
    pltpu.sync_copy(degv, dis_hbm.at[pl.ds(c * PAD + myrow, SLICE)])

    for t in range(SLICE // XC):
        r0 = myrow + t * XC
        pltpu.sync_copy(xpad_hbm.at[c, pl.ds(r0, XC)], xb)

        def xrow(lr, _):
            dv = plsc.load_gather(degv, [jnp.full((16,), t * XC + lr, jnp.int32)])
            for q in range(D // 16):
                xb[lr, pl.ds(q * 16, 16)] = xb[lr, pl.ds(q * 16, 16)] * dv
            return 0
        lax.fori_loop(0, XC, xrow, 0)
        pltpu.sync_copy(xb, z0_hbm.at[pl.ds(c * PAD + r0, XC)])


def _layer_body(z_hbm, esrc_hbm, edst_hbm, ecnt_hbm, dis_hbm, tin_hbm,
                *refs, last):
    if last:
        (tout_hbm, idxs, idxd, rows, cntb, disv, sb, tb, sacc, sem) = refs
        znext_hbm = None
    else:
        (znext_hbm, tout_hbm, idxs, idxd, rows, cntb, disv, sb, tb, sacc, sem) = refs
    c = lax.axis_index("c")
    s = lax.axis_index("s")
    w = c * 16 + s
    myrow = s * SLICE

    _zero_rows(sb, XC)
    for t in range(SLICE // XC):
        pltpu.sync_copy(sb, sacc.at[pl.ds(myrow + t * XC, XC)])
    plsc.subcore_barrier()

    pltpu.sync_copy(ecnt_hbm.at[pl.ds(w * 16, 16)], cntb)
    nwin = cntb[pl.ds(0, 16)][0] // W

    def win(g, _):
        pltpu.sync_copy(esrc_hbm.at[pl.ds(w * CAP + g * W, W)], idxs)
        pltpu.sync_copy(edst_hbm.at[pl.ds(w * CAP + g * W, W)], idxd)
        pltpu.async_copy(z_hbm.at[idxs], rows, sem).wait()
        pltpu.sync_copy(rows, sacc.at[idxd], add=True)
        return 0
    lax.fori_loop(0, nwin, win, 0)
    plsc.subcore_barrier()

    pltpu.sync_copy(dis_hbm.at[pl.ds(c * PAD + myrow, SLICE)], disv)
    for t in range(SLICE // XC):
        r0 = myrow + t * XC
        pltpu.sync_copy(sacc.at[pl.ds(r0, XC)], sb)
        pltpu.sync_copy(tin_hbm.at[c, pl.ds(r0, XC)], tb)

        def rbody(lr, _):
            dv = plsc.load_gather(disv, [jnp.full((16,), t * XC + lr, jnp.int32)])
            d2 = dv * dv
            for q in range(D // 16):
                sv = sb[lr, pl.ds(q * 16, 16)]
                tv = tb[lr, pl.ds(q * 16, 16)]
                nt = tv + dv * sv
                if last:
                    nt = nt * 0.25
                tb[lr, pl.ds(q * 16, 16)] = nt
                if not last:
                    sb[lr, pl.ds(q * 16, 16)] = d2 * sv
            return 0
        lax.fori_loop(0, XC, rbody, 0)
        if not last:
            pltpu.sync_copy(sb, znext_hbm.at[pl.ds(c * PAD + r0, XC)])
        pltpu.sync_copy(tb, tout_hbm.at[c, pl.ds(r0, XC)])


_params = pltpu.CompilerParams(
    needs_layout_passes=False, use_tc_tiling_on_sc=False)


@functools.lru_cache(maxsize=None)
def _route():
  return pl.kernel(
    _route_body, mesh=_mesh(), compiler_params=_params,
    out_type=(
        jax.ShapeDtypeStruct((32 * CAP,), jnp.int32),
        jax.ShapeDtypeStruct((32 * CAP,), jnp.int32),
        jax.ShapeDtypeStruct((32 * 16,), jnp.int32),
        jax.ShapeDtypeStruct((2 * PAD,), jnp.float32),
        jax.ShapeDtypeStruct((2 * PAD, D), jnp.float32),
    ),
    scratch_types=[
        pltpu.VMEM((CHUNK,), jnp.int32),
        pltpu.VMEM((CHUNK,), jnp.int32),
        pltpu.VMEM((CAP,), jnp.int32),
        pltpu.VMEM((CAP,), jnp.int32),
        pltpu.VMEM((W,), jnp.int32),
        pltpu.VMEM((W,), jnp.float32),
        pltpu.VMEM((SLICE,), jnp.float32),
        pltpu.VMEM((16,), jnp.int32),
        pltpu.VMEM((XC, D), jnp.float32),
        pltpu.VMEM_SHARED((PAD,), jnp.float32),
        pltpu.SemaphoreType.DMA,
    ],
  )


def _layer_scratch():
  return [
    pltpu.VMEM((W,), jnp.int32),
    pltpu.VMEM((W,), jnp.int32),
    pltpu.VMEM((W, D), jnp.float32),
    pltpu.VMEM((16,), jnp.int32),
    pltpu.VMEM((SLICE,), jnp.float32),
    pltpu.VMEM((XC, D), jnp.float32),
    pltpu.VMEM((XC, D), jnp.float32),
    pltpu.VMEM_SHARED((PAD, D), jnp.float32),
    pltpu.SemaphoreType.DMA,
  ]


@functools.lru_cache(maxsize=None)
def _layer_mid():
  return pl.kernel(
    functools.partial(_layer_body, last=False), mesh=_mesh(),
    compiler_params=_params,
    out_type=(
        jax.ShapeDtypeStruct((2 * PAD, D), jnp.float32),
        jax.ShapeDtypeStruct((2, PAD, D), jnp.float32),
    ),
    scratch_types=_layer_scratch(),
  )


@functools.lru_cache(maxsize=None)
def _layer_last():
  return pl.kernel(
    functools.partial(_layer_body, last=True), mesh=_mesh(),
    compiler_params=_params,
    out_type=jax.ShapeDtypeStruct((2, PAD, D), jnp.float32),
    scratch_types=_layer_scratch(),
  )


def kernel(user_emb, movie_emb, edge_index):
    ei = edge_index.astype(jnp.int32)
    src = ei[0]
    dst = ei[1]
    x = jnp.concatenate([user_emb, movie_emb], axis=0)
    xpad = jnp.zeros((2, PAD, D), jnp.float32).at[:, :HALF, :].set(
        x.reshape(2, HALF, D))
    esrc, edst, ecnt, dis, z = _route()(src, dst, xpad)
    t = xpad
    z, t = _layer_mid()(z, esrc, edst, ecnt, dis, t)
    z, t = _layer_mid()(z, esrc, edst, ecnt, dis, t)
    t = _layer_last()(z, esrc, edst, ecnt, dis, t)
    return (t[0, :HALF], t[1, :HALF])


# speedup vs baseline: 16.5671x; 1.3434x over previous
import functools

import jax
import jax.numpy as jnp
from jax import lax
from jax.experimental import pallas as pl
from jax.experimental.pallas import tpu as pltpu
from jax.experimental.pallas import tpu_sc as plsc

N_USERS = 25000
N_MOVIES = 25000
N_NODES = 50000
D = 64
HALF = 25000
PAD = 25088
SLICE = PAD // 16
XC = SLICE // 28
W = 128
PADW = 2 * W
E = 800000
ESL = E // 16
CAP = 50432
CHUNK = 2000
NTRASH = PAD - HALF

@functools.lru_cache(maxsize=None)
def _mesh():
    return plsc.VectorSubcoreMesh(core_axis_name="c", subcore_axis_name="s")


def _zero_rows(buf, nrows):
    zv = jnp.zeros((16,), jnp.float32)

    def body(r, _):
        for q in range(D // 16):
            buf[r, pl.ds(q * 16, 16)] = zv
        return 0

    lax.fori_loop(0, nrows, body, 0)


def _route_body(src_hbm, dst_hbm, xpad_hbm,
                esrc_hbm, edst_hbm, ecnt_hbm, dis_hbm, z0_hbm,
                srcb, dstb, lsrc, ldst, idxb, onesb, degv, cntb, xb, sdeg, sem):
    c = lax.axis_index("c")
    s = lax.axis_index("s")
    w = c * 16 + s
    base = c * HALF
    lanes = lax.iota(jnp.int32, 16)
    myrow = s * SLICE

    def zbody(k, _):
        degv[pl.ds(k * 16, 16)] = jnp.zeros((16,), jnp.float32)
        return 0
    lax.fori_loop(0, SLICE // 16, zbody, 0)
    pltpu.sync_copy(degv, sdeg.at[pl.ds(myrow, SLICE)])
    plsc.subcore_barrier()

    e0 = s * ESL

    def chunk_body(i, cnt):
        off = e0 + i * CHUNK
        pltpu.sync_copy(src_hbm.at[pl.ds(off, CHUNK)], srcb)
        pltpu.sync_copy(dst_hbm.at[pl.ds(off, CHUNK)], dstb)

        def vec_body(k, cnt):
            d = dstb[pl.ds(k * 16, 16)]
            sv = srcb[pl.ds(k * 16, 16)]
            inh = (d >= base) & (d < base + HALF)
            dl = d - base
            sp = sv + (PAD - HALF) * (sv >= HALF).astype(jnp.int32)
            cs = plsc.cumsum(inh.astype(jnp.int32))
            pos = cnt + cs - 1
            plsc.store_scatter(lsrc, [pos], sp, mask=inh)
            plsc.store_scatter(ldst, [pos], dl, mask=inh)
            return cnt + cs[15]

        return lax.fori_loop(0, CHUNK // 16, vec_body, cnt)

    cnt = lax.fori_loop(0, ESL // CHUNK, chunk_body, 0)
    cntp = ((cnt + PADW - 1) // PADW) * PADW

    for j in range(PADW // 16):
        lsrc[pl.ds(cnt + j * 16, 16)] = s * 97 + j * 16 + lanes
        ldst[pl.ds(cnt + j * 16, 16)] = HALF + ((j * 16 + lanes) % NTRASH)

    pltpu.sync_copy(lsrc, esrc_hbm.at[pl.ds(w * CAP, CAP)])
    pltpu.sync_copy(ldst, edst_hbm.at[pl.ds(w * CAP, CAP)])
    cntb[pl.ds(0, 16)] = jnp.full((16,), cntp, jnp.int32)
    pltpu.sync_copy(cntb, ecnt_hbm.at[pl.ds(w * 16, 16)])

    def obody(j, _):
        onesb[pl.ds(j * 16, 16)] = jnp.ones((16,), jnp.float32)
        return 0
    lax.fori_loop(0, W // 16, obody, 0)
    nwin = cntp // W

    def deg_body(g, _):
        def cp(j, _):
            idxb[pl.ds(j * 16, 16)] = ldst[pl.ds(g * W + j * 16, 16)]
            return 0
        lax.fori_loop(0, W // 16, cp, 0)
        pltpu.sync_copy(onesb, sdeg.at[idxb], add=True)
        return 0
    lax.fori_loop(0, nwin, deg_body, 0)
    plsc.subcore_barrier()

    pltpu.sync_copy(sdeg.at[pl.ds(myrow, SLICE)], degv)

    def nbody(k, _):
        dgv = degv[pl.ds(k * 16, 16)]
        bi = lax.bitcast_convert_type(dgv, jnp.int32)
        y = lax.bitcast_convert_type(
            jnp.int32(0x5F3759DF) - lax.shift_right_logical(bi, 1), jnp.float32)
        for _ in range(3):
            y = y * (1.5 - 0.5 * dgv * y * y)
        degv[pl.ds(k * 16, 16)] = jnp.where(dgv >= 1.0, y, 0.0)
        return 0
    lax.fori_loop(0, SLICE // 16, nbody, 0)
    pltpu.sync_copy(degv, dis_hbm.at[pl.ds(c * PAD + myrow, SLICE)])

    for t in range(SLICE // XC):
        r0 = myrow + t * XC
        pltpu.sync_copy(xpad_hbm.at[c, pl.ds(r0, XC)], xb)

        def xrow(lr, _):
            dv = plsc.load_gather(degv, [jnp.full((16,), t * XC + lr, jnp.int32)])
            for q in range(D // 16):
                xb[lr, pl.ds(q * 16, 16)] = xb[lr, pl.ds(q * 16, 16)] * dv
            return 0
        lax.fori_loop(0, XC, xrow, 0)
        pltpu.sync_copy(xb, z0_hbm.at[pl.ds(c * PAD + r0, XC)])


def _layer_body(z_hbm, esrc_hbm, edst_hbm, ecnt_hbm, dis_hbm, tin_hbm,
                *refs, last):
    if last:
        (tout_hbm, idxs0, idxs1, idxd0, idxd1, rows, cntb, disv, sb, tb,
         sacc, semg0, semg1, sems0, sems1) = refs
        znext_hbm = None
    else:
        (znext_hbm, tout_hbm, idxs0, idxs1, idxd0, idxd1, rows, cntb, disv,
         sb, tb, sacc, semg0, semg1, sems0, sems1) = refs
    idxs = (idxs0, idxs1)
    idxd = (idxd0, idxd1)
    semg = (semg0, semg1)
    sems = (sems0, sems1)
    c = lax.axis_index("c")
    s = lax.axis_index("s")
    w = c * 16 + s
    myrow = s * SLICE

    _zero_rows(sb, XC)
    for t in range(SLICE // XC):
        pltpu.sync_copy(sb, sacc.at[pl.ds(myrow + t * XC, XC)])
    plsc.subcore_barrier()

    pltpu.sync_copy(ecnt_hbm.at[pl.ds(w * 16, 16)], cntb)
    nwin = cntb[pl.ds(0, 16)][0] // W

    def _load_idx(b, g):
        pltpu.sync_copy(esrc_hbm.at[pl.ds(w * CAP + g * W, W)], idxs[b])
        pltpu.sync_copy(edst_hbm.at[pl.ds(w * CAP + g * W, W)], idxd[b])

    @pl.when(nwin > 0)
    def _():
        _load_idx(0, 0)
        pltpu.async_copy(z_hbm.at[idxs[0]], rows.at[0], semg[0])

    def pair(p, _):
        for b in range(2):
            g = 2 * p + b
            nb = 1 - b

            @pl.when(g + 1 < nwin)
            def _():
                @pl.when(g >= 1)
                def _():
                    pltpu.make_async_copy(
                        rows.at[nb], sacc.at[idxd[nb]], sems[nb]).wait()
                _load_idx(nb, g + 1)
                pltpu.async_copy(z_hbm.at[idxs[nb]], rows.at[nb], semg[nb])

            pltpu.make_async_copy(z_hbm.at[idxs[b]], rows.at[b], semg[b]).wait()
            pltpu.async_copy(rows.at[b], sacc.at[idxd[b]], sems[b], add=True)
        return 0

    lax.fori_loop(0, nwin // 2, pair, 0)

    @pl.when(nwin > 0)
    def _():
        pltpu.make_async_copy(rows.at[0], sacc.at[idxd[0]], sems[0]).wait()
        pltpu.make_async_copy(rows.at[1], sacc.at[idxd[1]], sems[1]).wait()
    plsc.subcore_barrier()

    pltpu.sync_copy(dis_hbm.at[pl.ds(c * PAD + myrow, SLICE)], disv)
    for t in range(SLICE // XC):
        r0 = myrow + t * XC
        pltpu.sync_copy(sacc.at[pl.ds(r0, XC)], sb)
        pltpu.sync_copy(tin_hbm.at[c, pl.ds(r0, XC)], tb)

        def rbody(lr, _):
            dv = plsc.load_gather(disv, [jnp.full((16,), t * XC + lr, jnp.int32)])
            d2 = dv * dv
            for q in range(D // 16):
                sv = sb[lr, pl.ds(q * 16, 16)]
                tv = tb[lr, pl.ds(q * 16, 16)]
                nt = tv + dv * sv
                if last:
                    nt = nt * 0.25
                tb[lr, pl.ds(q * 16, 16)] = nt
                if not last:
                    sb[lr, pl.ds(q * 16, 16)] = d2 * sv
            return 0
        lax.fori_loop(0, XC, rbody, 0)
        if not last:
            pltpu.sync_copy(sb, znext_hbm.at[pl.ds(c * PAD + r0, XC)])
        pltpu.sync_copy(tb, tout_hbm.at[c, pl.ds(r0, XC)])


_params = pltpu.CompilerParams(
    needs_layout_passes=False, use_tc_tiling_on_sc=False)


@functools.lru_cache(maxsize=None)
def _route():
  return pl.kernel(
    _route_body, mesh=_mesh(), compiler_params=_params,
    out_type=(
        jax.ShapeDtypeStruct((32 * CAP,), jnp.int32),
        jax.ShapeDtypeStruct((32 * CAP,), jnp.int32),
        jax.ShapeDtypeStruct((32 * 16,), jnp.int32),
        jax.ShapeDtypeStruct((2 * PAD,), jnp.float32),
        jax.ShapeDtypeStruct((2 * PAD, D), jnp.float32),
    ),
    scratch_types=[
        pltpu.VMEM((CHUNK,), jnp.int32),
        pltpu.VMEM((CHUNK,), jnp.int32),
        pltpu.VMEM((CAP,), jnp.int32),
        pltpu.VMEM((CAP,), jnp.int32),
        pltpu.VMEM((W,), jnp.int32),
        pltpu.VMEM((W,), jnp.float32),
        pltpu.VMEM((SLICE,), jnp.float32),
        pltpu.VMEM((16,), jnp.int32),
        pltpu.VMEM((XC, D), jnp.float32),
        pltpu.VMEM_SHARED((PAD,), jnp.float32),
        pltpu.SemaphoreType.DMA,
    ],
  )


def _layer_scratch():
  return [
    pltpu.VMEM((W,), jnp.int32),
    pltpu.VMEM((W,), jnp.int32),
    pltpu.VMEM((W,), jnp.int32),
    pltpu.VMEM((W,), jnp.int32),
    pltpu.VMEM((2, W, D), jnp.float32),
    pltpu.VMEM((16,), jnp.int32),
    pltpu.VMEM((SLICE,), jnp.float32),
    pltpu.VMEM((XC, D), jnp.float32),
    pltpu.VMEM((XC, D), jnp.float32),
    pltpu.VMEM_SHARED((PAD, D), jnp.float32),
    pltpu.SemaphoreType.DMA,
    pltpu.SemaphoreType.DMA,
    pltpu.SemaphoreType.DMA,
    pltpu.SemaphoreType.DMA,
  ]


@functools.lru_cache(maxsize=None)
def _layer_mid():
  return pl.kernel(
    functools.partial(_layer_body, last=False), mesh=_mesh(),
    compiler_params=_params,
    out_type=(
        jax.ShapeDtypeStruct((2 * PAD, D), jnp.float32),
        jax.ShapeDtypeStruct((2, PAD, D), jnp.float32),
    ),
    scratch_types=_layer_scratch(),
  )


@functools.lru_cache(maxsize=None)
def _layer_last():
  return pl.kernel(
    functools.partial(_layer_body, last=True), mesh=_mesh(),
    compiler_params=_params,
    out_type=jax.ShapeDtypeStruct((2, PAD, D), jnp.float32),
    scratch_types=_layer_scratch(),
  )


def kernel(user_emb, movie_emb, edge_index):
    ei = edge_index.astype(jnp.int32)
    src = ei[0]
    dst = ei[1]
    x = jnp.concatenate([user_emb, movie_emb], axis=0)
    xpad = jnp.zeros((2, PAD, D), jnp.float32).at[:, :HALF, :].set(
        x.reshape(2, HALF, D))
    esrc, edst, ecnt, dis, z = _route()(src, dst, xpad)
    t = xpad
    z, t = _layer_mid()(z, esrc, edst, ecnt, dis, t)
    z, t = _layer_mid()(z, esrc, edst, ecnt, dis, t)
    t = _layer_last()(z, esrc, edst, ecnt, dis, t)
    return (t[0, :HALF], t[1, :HALF])


# speedup vs baseline: 27.9194x; 1.6852x over previous
import functools

import jax
import jax.numpy as jnp
from jax import lax
from jax.experimental import pallas as pl
from jax.experimental.pallas import tpu as pltpu
from jax.experimental.pallas import tpu_sc as plsc

N_USERS = 25000
N_MOVIES = 25000
N_NODES = 50000
D = 64
HALF = 25000
PAD = 25088
SLICE = PAD // 16
XC = SLICE // 28
W = 128
PADW = 4 * W
E = 800000
ESL = E // 16
CAP = 50688
CHUNK = 2000
NTRASH = PAD - HALF

@functools.lru_cache(maxsize=None)
def _mesh():
    return plsc.VectorSubcoreMesh(core_axis_name="c", subcore_axis_name="s")


def _zero_rows(buf, nrows):
    zv = jnp.zeros((16,), jnp.float32)

    def body(r, _):
        for q in range(D // 16):
            buf[r, pl.ds(q * 16, 16)] = zv
        return 0

    lax.fori_loop(0, nrows, body, 0)


def _route_body(src_hbm, dst_hbm, xpad_hbm,
                esrc_hbm, edst_hbm, ecnt_hbm, dis_hbm, z0_hbm,
                srcb, dstb, lsrc, ldst, idxb, onesb, degv, cntb, xb, sdeg, sem):
    c = lax.axis_index("c")
    s = lax.axis_index("s")
    w = c * 16 + s
    base = c * HALF
    lanes = lax.iota(jnp.int32, 16)
    myrow = s * SLICE

    def zbody(k, _):
        degv[pl.ds(k * 16, 16)] = jnp.zeros((16,), jnp.float32)
        return 0
    lax.fori_loop(0, SLICE // 16, zbody, 0)
    pltpu.sync_copy(degv, sdeg.at[pl.ds(myrow, SLICE)])
    plsc.subcore_barrier()

    e0 = s * ESL

    def chunk_body(i, cnt):
        off = e0 + i * CHUNK
        pltpu.sync_copy(src_hbm.at[pl.ds(off, CHUNK)], srcb)
        pltpu.sync_copy(dst_hbm.at[pl.ds(off, CHUNK)], dstb)

        def vec_body(k, cnt):
            d = dstb[pl.ds(k * 16, 16)]
            sv = srcb[pl.ds(k * 16, 16)]
            inh = (d >= base) & (d < base + HALF)
            dl = d - base
            sp = sv + (PAD - HALF) * (sv >= HALF).astype(jnp.int32)
            cs = plsc.cumsum(inh.astype(jnp.int32))
            pos = cnt + cs - 1
            plsc.store_scatter(lsrc, [pos], sp, mask=inh)
            plsc.store_scatter(ldst, [pos], dl, mask=inh)
            return cnt + cs[15]

        return lax.fori_loop(0, CHUNK // 16, vec_body, cnt)

    cnt = lax.fori_loop(0, ESL // CHUNK, chunk_body, 0)
    cntp = ((cnt + PADW - 1) // PADW) * PADW

    for j in range(PADW // 16):
        lsrc[pl.ds(cnt + j * 16, 16)] = s * 97 + j * 16 + lanes
        ldst[pl.ds(cnt + j * 16, 16)] = HALF + ((j * 16 + lanes) % NTRASH)

    pltpu.sync_copy(lsrc, esrc_hbm.at[pl.ds(w * CAP, CAP)])
    pltpu.sync_copy(ldst, edst_hbm.at[pl.ds(w * CAP, CAP)])
    cntb[pl.ds(0, 16)] = jnp.full((16,), cntp, jnp.int32)
    pltpu.sync_copy(cntb, ecnt_hbm.at[pl.ds(w * 16, 16)])

    def obody(j, _):
        onesb[pl.ds(j * 16, 16)] = jnp.ones((16,), jnp.float32)
        return 0
    lax.fori_loop(0, W // 16, obody, 0)
    nwin = cntp // W

    def deg_body(g, _):
        def cp(j, _):
            idxb[pl.ds(j * 16, 16)] = ldst[pl.ds(g * W + j * 16, 16)]
            return 0
        lax.fori_loop(0, W // 16, cp, 0)
        pltpu.sync_copy(onesb, sdeg.at[idxb], add=True)
        return 0
    lax.fori_loop(0, nwin, deg_body, 0)
    plsc.subcore_barrier()

    pltpu.sync_copy(sdeg.at[pl.ds(myrow, SLICE)], degv)

    def nbody(k, _):
        dgv = degv[pl.ds(k * 16, 16)]
        bi = lax.bitcast_convert_type(dgv, jnp.int32)
        y = lax.bitcast_convert_type(
            jnp.int32(0x5F3759DF) - lax.shift_right_logical(bi, 1), jnp.float32)
        for _ in range(3):
            y = y * (1.5 - 0.5 * dgv * y * y)
        degv[pl.ds(k * 16, 16)] = jnp.where(dgv >= 1.0, y, 0.0)
        return 0
    lax.fori_loop(0, SLICE // 16, nbody, 0)
    pltpu.sync_copy(degv, dis_hbm.at[pl.ds(c * PAD + myrow, SLICE)])

    for t in range(SLICE // XC):
        r0 = myrow + t * XC
        pltpu.sync_copy(xpad_hbm.at[c, pl.ds(r0, XC)], xb)

        def xrow(lr, _):
            dv = plsc.load_gather(degv, [jnp.full((16,), t * XC + lr, jnp.int32)])
            for q in range(D // 16):
                xb[lr, pl.ds(q * 16, 16)] = xb[lr, pl.ds(q * 16, 16)] * dv
            return 0
        lax.fori_loop(0, XC, xrow, 0)
        pltpu.sync_copy(xb, z0_hbm.at[pl.ds(c * PAD + r0, XC)])


def _layer_body(z_hbm, esrc_hbm, edst_hbm, ecnt_hbm, dis_hbm, tin_hbm,
                *refs, last):
    if last:
        (tout_hbm, idxs0, idxs1, idxs2, idxs3, idxd0, idxd1, idxd2, idxd3,
         rows, cntb, disv, sb, tb, sacc,
         semg0, semg1, sems0, sems1, semi0, semi1, semi2, semi3,
         semz, semt0, semt1, semwz0, semwz1, semwt0, semwt1) = refs
        znext_hbm = None
    else:
        (znext_hbm, tout_hbm, idxs0, idxs1, idxs2, idxs3, idxd0, idxd1,
         idxd2, idxd3, rows, cntb, disv, sb, tb, sacc,
         semg0, semg1, sems0, sems1, semi0, semi1, semi2, semi3,
         semz, semt0, semt1, semwz0, semwz1, semwt0, semwt1) = refs
    idxs = (idxs0, idxs1, idxs2, idxs3)
    idxd = (idxd0, idxd1, idxd2, idxd3)
    semg = (semg0, semg1)
    sems = (sems0, sems1)
    semi = (semi0, semi1, semi2, semi3)
    semt = (semt0, semt1)
    semwz = (semwz0, semwz1)
    semwt = (semwt0, semwt1)
    c = lax.axis_index("c")
    s = lax.axis_index("s")
    w = c * 16 + s
    myrow = s * SLICE
    NCH = SLICE // XC

    _zero_rows(sb, XC)
    for t in range(NCH):
        pltpu.async_copy(sb, sacc.at[pl.ds(myrow + t * XC, XC)], semz)
    for t in range(NCH):
        pltpu.make_async_copy(sb, sacc.at[pl.ds(myrow + t * XC, XC)], semz).wait()
    plsc.subcore_barrier()

    pltpu.sync_copy(ecnt_hbm.at[pl.ds(w * 16, 16)], cntb)
    nwin = cntb[pl.ds(0, 16)][0] // W

    def _idx_copy(k, g, start):
        a = pltpu.make_async_copy(
            esrc_hbm.at[pl.ds(w * CAP + g * W, W)], idxs[k], semi[k])
        b = pltpu.make_async_copy(
            edst_hbm.at[pl.ds(w * CAP + g * W, W)], idxd[k], semi[k])
        if start:
            a.start()
            b.start()
        else:
            a.wait()
            b.wait()

    @pl.when(nwin > 0)
    def _():
        _idx_copy(0, 0, True)
        _idx_copy(1, 1, True)
        _idx_copy(0, 0, False)
        pltpu.async_copy(z_hbm.at[idxs[0]], rows.at[0], semg[0])

    def quad(q, _):
        for j in range(4):
            g = 4 * q + j
            b = j & 1
            nb = 1 - b
            kn = (j + 1) % 4

            @pl.when((g + 1 < nwin) & (g >= 1))
            def _():
                pltpu.make_async_copy(
                    rows.at[nb], sacc.at[idxd[(j + 3) % 4]], sems[nb]).wait()

            @pl.when(g + 2 < nwin)
            def _():
                _idx_copy((j + 2) % 4, g + 2, True)

            @pl.when(g + 1 < nwin)
            def _():
                _idx_copy(kn, g + 1, False)
                pltpu.async_copy(z_hbm.at[idxs[kn]], rows.at[nb], semg[nb])

            pltpu.make_async_copy(z_hbm.at[idxs[j]], rows.at[b], semg[b]).wait()
            pltpu.async_copy(rows.at[b], sacc.at[idxd[j]], sems[b], add=True)
        return 0

    lax.fori_loop(0, nwin // 4, quad, 0)

    @pl.when(nwin > 0)
    def _():
        pltpu.make_async_copy(rows.at[0], sacc.at[idxd[2]], sems[0]).wait()
        pltpu.make_async_copy(rows.at[1], sacc.at[idxd[3]], sems[1]).wait()
    plsc.subcore_barrier()

    pltpu.sync_copy(dis_hbm.at[pl.ds(c * PAD + myrow, SLICE)], disv)

    def _tin_copy(bb, t, start):
        a = pltpu.make_async_copy(
            tin_hbm.at[c, pl.ds(myrow + t * XC, XC)], tb.at[bb], semt[bb])
        a.start() if start else a.wait()

    def _st_t(bb, t, start):
        a = pltpu.make_async_copy(
            rows.at[bb, pl.ds(0, XC)],
            tout_hbm.at[c, pl.ds(myrow + t * XC, XC)], semwt[bb])
        a.start() if start else a.wait()

    def _st_z(bb, t, start):
        a = pltpu.make_async_copy(
            rows.at[bb, pl.ds(64, XC)],
            znext_hbm.at[pl.ds(c * PAD + myrow + t * XC, XC)], semwz[bb])
        a.start() if start else a.wait()

    _tin_copy(0, 0, True)
    _tin_copy(1, 1, True)

    def spair(tp, _):
        for bb in range(2):
            t = 2 * tp + bb

            @pl.when(t >= 2)
            def _():
                _st_t(bb, t - 2, False)
                if not last:
                    _st_z(bb, t - 2, False)

            pltpu.sync_copy(sacc.at[pl.ds(myrow + t * XC, XC)], sb)
            _tin_copy(bb, t, False)

            def rbody(lr, _):
                dv = plsc.load_gather(
                    disv, [jnp.full((16,), t * XC + lr, jnp.int32)])
                d2 = dv * dv
                for qq in range(D // 16):
                    sv = sb[lr, pl.ds(qq * 16, 16)]
                    tv = tb[bb, lr, pl.ds(qq * 16, 16)]
                    nt = tv + dv * sv
                    if last:
                        nt = nt * 0.25
                    rows[bb, lr, pl.ds(qq * 16, 16)] = nt
                    if not last:
                        rows[bb, 64 + lr, pl.ds(qq * 16, 16)] = d2 * sv
                return 0
            lax.fori_loop(0, XC, rbody, 0)
            _st_t(bb, t, True)
            if not last:
                _st_z(bb, t, True)

            @pl.when(t + 2 < NCH)
            def _():
                _tin_copy(bb, t + 2, True)
        return 0

    lax.fori_loop(0, NCH // 2, spair, 0)
    _st_t(0, NCH - 2, False)
    _st_t(1, NCH - 1, False)
    if not last:
        _st_z(0, NCH - 2, False)
        _st_z(1, NCH - 1, False)


_params = pltpu.CompilerParams(
    needs_layout_passes=False, use_tc_tiling_on_sc=False)


@functools.lru_cache(maxsize=None)
def _route():
  return pl.kernel(
    _route_body, mesh=_mesh(), compiler_params=_params,
    out_type=(
        jax.ShapeDtypeStruct((32 * CAP,), jnp.int32),
        jax.ShapeDtypeStruct((32 * CAP,), jnp.int32),
        jax.ShapeDtypeStruct((32 * 16,), jnp.int32),
        jax.ShapeDtypeStruct((2 * PAD,), jnp.float32),
        jax.ShapeDtypeStruct((2 * PAD, D), jnp.float32),
    ),
    scratch_types=[
        pltpu.VMEM((CHUNK,), jnp.int32),
        pltpu.VMEM((CHUNK,), jnp.int32),
        pltpu.VMEM((CAP,), jnp.int32),
        pltpu.VMEM((CAP,), jnp.int32),
        pltpu.VMEM((W,), jnp.int32),
        pltpu.VMEM((W,), jnp.float32),
        pltpu.VMEM((SLICE,), jnp.float32),
        pltpu.VMEM((16,), jnp.int32),
        pltpu.VMEM((XC, D), jnp.float32),
        pltpu.VMEM_SHARED((PAD,), jnp.float32),
        pltpu.SemaphoreType.DMA,
    ],
  )


def _layer_scratch():
  return (
    [pltpu.VMEM((W,), jnp.int32)] * 8
    + [
      pltpu.VMEM((2, W, D), jnp.float32),
      pltpu.VMEM((16,), jnp.int32),
      pltpu.VMEM((SLICE,), jnp.float32),
      pltpu.VMEM((XC, D), jnp.float32),
      pltpu.VMEM((2, XC, D), jnp.float32),
      pltpu.VMEM_SHARED((PAD, D), jnp.float32),
    ]
    + [pltpu.SemaphoreType.DMA] * 15
  )


@functools.lru_cache(maxsize=None)
def _layer_mid():
  return pl.kernel(
    functools.partial(_layer_body, last=False), mesh=_mesh(),
    compiler_params=_params,
    out_type=(
        jax.ShapeDtypeStruct((2 * PAD, D), jnp.float32),
        jax.ShapeDtypeStruct((2, PAD, D), jnp.float32),
    ),
    scratch_types=_layer_scratch(),
  )


@functools.lru_cache(maxsize=None)
def _layer_last():
  return pl.kernel(
    functools.partial(_layer_body, last=True), mesh=_mesh(),
    compiler_params=_params,
    out_type=jax.ShapeDtypeStruct((2, PAD, D), jnp.float32),
    scratch_types=_layer_scratch(),
  )


def kernel(user_emb, movie_emb, edge_index):
    ei = edge_index.astype(jnp.int32)
    src = ei[0]
    dst = ei[1]
    x = jnp.concatenate([user_emb, movie_emb], axis=0)
    xpad = jnp.zeros((2, PAD, D), jnp.float32).at[:, :HALF, :].set(
        x.reshape(2, HALF, D))
    esrc, edst, ecnt, dis, z = _route()(src, dst, xpad)
    t = xpad
    z, t = _layer_mid()(z, esrc, edst, ecnt, dis, t)
    z, t = _layer_mid()(z, esrc, edst, ecnt, dis, t)
    t = _layer_last()(z, esrc, edst, ecnt, dis, t)
    return (t[0, :HALF], t[1, :HALF])


# speedup vs baseline: 30.5049x; 1.0926x over previous
import functools

import jax
import jax.numpy as jnp
from jax import lax
from jax.experimental import pallas as pl
from jax.experimental.pallas import tpu as pltpu
from jax.experimental.pallas import tpu_sc as plsc

N_USERS = 25000
N_MOVIES = 25000
N_NODES = 50000
D = 64
HALF = 25000
PAD = 25088
SLICE = PAD // 16
XC = SLICE // 28
W = 128
PADW = 4 * W
E = 800000
ESL = E // 16
CAP = 50688
CHUNK = 2000
NTRASH = PAD - HALF

@functools.lru_cache(maxsize=None)
def _mesh():
    return plsc.VectorSubcoreMesh(core_axis_name="c", subcore_axis_name="s")


def _zero_rows(buf, nrows):
    zv = jnp.zeros((16,), jnp.float32)

    def body(r, _):
        for q in range(D // 16):
            buf[r, pl.ds(q * 16, 16)] = zv
        return 0

    lax.fori_loop(0, nrows, body, 0)


def _route_body(src_hbm, dst_hbm, xpad_hbm,
                esrc_hbm, edst_hbm, ecnt_hbm, dis_hbm, z0_hbm,
                srcb, dstb, lsrc, ldst, idxb0, idxb1, onesb, degv, cntb, xb,
                sdeg, semc, semd0, semd1, semx0, semx1, semw0, semw1):
    idxb = (idxb0, idxb1)
    semd = (semd0, semd1)
    semx = (semx0, semx1)
    semw = (semw0, semw1)
    c = lax.axis_index("c")
    s = lax.axis_index("s")
    w = c * 16 + s
    base = c * HALF
    lanes = lax.iota(jnp.int32, 16)
    myrow = s * SLICE
    NCHK = ESL // CHUNK

    def zbody(k, _):
        degv[pl.ds(k * 16, 16)] = jnp.zeros((16,), jnp.float32)
        return 0
    lax.fori_loop(0, SLICE // 16, zbody, 0)
    pltpu.sync_copy(degv, sdeg.at[pl.ds(myrow, SLICE)])
    plsc.subcore_barrier()

    e0 = s * ESL

    def _chunk_copy(bb, i, start):
        off = e0 + i * CHUNK
        a = pltpu.make_async_copy(
            src_hbm.at[pl.ds(off, CHUNK)], srcb.at[bb], semc)
        b = pltpu.make_async_copy(
            dst_hbm.at[pl.ds(off, CHUNK)], dstb.at[bb], semc)
        if start:
            a.start()
            b.start()
        else:
            a.wait()
            b.wait()

    _chunk_copy(0, 0, True)

    def chunk_pair(ip, cnt):
        for bb in range(2):
            i = 2 * ip + bb

            @pl.when(i + 1 < NCHK)
            def _():
                _chunk_copy(1 - bb, i + 1, True)
            _chunk_copy(bb, i, False)

            def vec_body(k, cnt):
                d = dstb[bb, pl.ds(k * 16, 16)]
                sv = srcb[bb, pl.ds(k * 16, 16)]
                inh = (d >= base) & (d < base + HALF)
                dl = d - base
                sp = sv + (PAD - HALF) * (sv >= HALF).astype(jnp.int32)
                cs = plsc.cumsum(inh.astype(jnp.int32))
                pos = cnt + cs - 1
                plsc.store_scatter(lsrc, [pos], sp, mask=inh)
                plsc.store_scatter(ldst, [pos], dl, mask=inh)
                return cnt + cs[15]

            cnt = lax.fori_loop(0, CHUNK // 16, vec_body, cnt)
        return cnt

    cnt = lax.fori_loop(0, NCHK // 2, chunk_pair, 0)
    if NCHK % 2:
        _chunk_copy(0, NCHK - 1, False)

        def vec_tail(k, cnt):
            d = dstb[0, pl.ds(k * 16, 16)]
            sv = srcb[0, pl.ds(k * 16, 16)]
            inh = (d >= base) & (d < base + HALF)
            dl = d - base
            sp = sv + (PAD - HALF) * (sv >= HALF).astype(jnp.int32)
            cs = plsc.cumsum(inh.astype(jnp.int32))
            pos = cnt + cs - 1
            plsc.store_scatter(lsrc, [pos], sp, mask=inh)
            plsc.store_scatter(ldst, [pos], dl, mask=inh)
            return cnt + cs[15]

        cnt = lax.fori_loop(0, CHUNK // 16, vec_tail, cnt)
    cntp = ((cnt + PADW - 1) // PADW) * PADW

    for j in range(PADW // 16):
        lsrc[pl.ds(cnt + j * 16, 16)] = s * 97 + j * 16 + lanes
        ldst[pl.ds(cnt + j * 16, 16)] = HALF + ((j * 16 + lanes) % NTRASH)

    pltpu.async_copy(lsrc, esrc_hbm.at[pl.ds(w * CAP, CAP)], semw0)
    pltpu.async_copy(ldst, edst_hbm.at[pl.ds(w * CAP, CAP)], semw1)
    cntb[pl.ds(0, 16)] = jnp.full((16,), cntp, jnp.int32)
    pltpu.sync_copy(cntb, ecnt_hbm.at[pl.ds(w * 16, 16)])

    def obody(j, _):
        onesb[pl.ds(j * 16, 16)] = jnp.ones((16,), jnp.float32)
        return 0
    lax.fori_loop(0, W // 16, obody, 0)
    nwin = cntp // W

    def _fill_idx(bb, g):
        def cp(j, _):
            idxb[bb][pl.ds(j * 16, 16)] = ldst[pl.ds(g * W + j * 16, 16)]
            return 0
        lax.fori_loop(0, W // 16, cp, 0)

    @pl.when(nwin > 0)
    def _():
        _fill_idx(0, 0)
        pltpu.async_copy(onesb, sdeg.at[idxb[0]], semd[0], add=True)

    def deg_pair(gp, _):
        for bb in range(2):
            g = 2 * gp + bb
            nb = 1 - bb

            @pl.when(g + 1 < nwin)
            def _():
                @pl.when(g >= 1)
                def _():
                    pltpu.make_async_copy(
                        onesb, sdeg.at[idxb[nb]], semd[nb]).wait()
                _fill_idx(nb, g + 1)
                pltpu.async_copy(onesb, sdeg.at[idxb[nb]], semd[nb], add=True)
        return 0
    lax.fori_loop(0, nwin // 2, deg_pair, 0)

    @pl.when(nwin > 0)
    def _():
        pltpu.make_async_copy(onesb, sdeg.at[idxb[0]], semd[0]).wait()
        pltpu.make_async_copy(onesb, sdeg.at[idxb[1]], semd[1]).wait()
    plsc.subcore_barrier()

    pltpu.sync_copy(sdeg.at[pl.ds(myrow, SLICE)], degv)

    def nbody(k, _):
        dgv = degv[pl.ds(k * 16, 16)]
        bi = lax.bitcast_convert_type(dgv, jnp.int32)
        y = lax.bitcast_convert_type(
            jnp.int32(0x5F3759DF) - lax.shift_right_logical(bi, 1), jnp.float32)
        for _ in range(3):
            y = y * (1.5 - 0.5 * dgv * y * y)
        degv[pl.ds(k * 16, 16)] = jnp.where(dgv >= 1.0, y, 0.0)
        return 0
    lax.fori_loop(0, SLICE // 16, nbody, 0)
    pltpu.sync_copy(degv, dis_hbm.at[pl.ds(c * PAD + myrow, SLICE)])

    NCH = SLICE // XC

    def _x_copy(bb, t, start):
        a = pltpu.make_async_copy(
            xpad_hbm.at[c, pl.ds(myrow + t * XC, XC)], xb.at[bb], semx[bb])
        a.start() if start else a.wait()

    def _z_store(bb, t, start):
        a = pltpu.make_async_copy(
            xb.at[bb], z0_hbm.at[pl.ds(c * PAD + myrow + t * XC, XC)], semw[bb])
        a.start() if start else a.wait()

    pltpu.make_async_copy(lsrc, esrc_hbm.at[pl.ds(w * CAP, CAP)], semw0).wait()
    pltpu.make_async_copy(ldst, edst_hbm.at[pl.ds(w * CAP, CAP)], semw1).wait()
    _x_copy(0, 0, True)
    _x_copy(1, 1, True)

    def zpair(tp, _):
        for bb in range(2):
            t = 2 * tp + bb

            @pl.when(t >= 2)
            def _():
                _z_store(bb, t - 2, False)
            _x_copy(bb, t, False)

            def xrow(lr, _):
                dv = plsc.load_gather(
                    degv, [jnp.full((16,), t * XC + lr, jnp.int32)])
                for q in range(D // 16):
                    xb[bb, lr, pl.ds(q * 16, 16)] = (
                        xb[bb, lr, pl.ds(q * 16, 16)] * dv)
                return 0
            lax.fori_loop(0, XC, xrow, 0)
            _z_store(bb, t, True)

            @pl.when(t + 2 < NCH)
            def _():
                _x_copy(bb, t + 2, True)
        return 0
    lax.fori_loop(0, NCH // 2, zpair, 0)
    _z_store(0, NCH - 2, False)
    _z_store(1, NCH - 1, False)


def _layer_body(z_hbm, esrc_hbm, edst_hbm, ecnt_hbm, dis_hbm, tin_hbm,
                *refs, last):
    if last:
        (tout_hbm, idxs0, idxs1, idxs2, idxs3, idxd0, idxd1, idxd2, idxd3,
         rows, cntb, disv, sb, tb, sacc,
         semg0, semg1, sems0, sems1, semi0, semi1, semi2, semi3,
         semz, semt0, semt1, semwz0, semwz1, semwt0, semwt1) = refs
        znext_hbm = None
    else:
        (znext_hbm, tout_hbm, idxs0, idxs1, idxs2, idxs3, idxd0, idxd1,
         idxd2, idxd3, rows, cntb, disv, sb, tb, sacc,
         semg0, semg1, sems0, sems1, semi0, semi1, semi2, semi3,
         semz, semt0, semt1, semwz0, semwz1, semwt0, semwt1) = refs
    idxs = (idxs0, idxs1, idxs2, idxs3)
    idxd = (idxd0, idxd1, idxd2, idxd3)
    semg = (semg0, semg1)
    sems = (sems0, sems1)
    semi = (semi0, semi1, semi2, semi3)
    semt = (semt0, semt1)
    semwz = (semwz0, semwz1)
    semwt = (semwt0, semwt1)
    c = lax.axis_index("c")
    s = lax.axis_index("s")
    w = c * 16 + s
    myrow = s * SLICE
    NCH = SLICE // XC

    _zero_rows(sb, XC)
    for t in range(NCH):
        pltpu.async_copy(sb, sacc.at[pl.ds(myrow + t * XC, XC)], semz)
    for t in range(NCH):
        pltpu.make_async_copy(sb, sacc.at[pl.ds(myrow + t * XC, XC)], semz).wait()
    plsc.subcore_barrier()

    pltpu.sync_copy(ecnt_hbm.at[pl.ds(w * 16, 16)], cntb)
    nwin = cntb[pl.ds(0, 16)][0] // W

    def _idx_copy(k, g, start):
        a = pltpu.make_async_copy(
            esrc_hbm.at[pl.ds(w * CAP + g * W, W)], idxs[k], semi[k])
        b = pltpu.make_async_copy(
            edst_hbm.at[pl.ds(w * CAP + g * W, W)], idxd[k], semi[k])
        if start:
            a.start()
            b.start()
        else:
            a.wait()
            b.wait()

    @pl.when(nwin > 0)
    def _():
        _idx_copy(0, 0, True)
        _idx_copy(1, 1, True)
        _idx_copy(0, 0, False)
        pltpu.async_copy(z_hbm.at[idxs[0]], rows.at[0], semg[0])

    def quad(q, _):
        for j in range(4):
            g = 4 * q + j
            b = j & 1
            nb = 1 - b
            kn = (j + 1) % 4

            @pl.when((g + 1 < nwin) & (g >= 1))
            def _():
                pltpu.make_async_copy(
                    rows.at[nb], sacc.at[idxd[(j + 3) % 4]], sems[nb]).wait()

            @pl.when(g + 2 < nwin)
            def _():
                _idx_copy((j + 2) % 4, g + 2, True)

            @pl.when(g + 1 < nwin)
            def _():
                _idx_copy(kn, g + 1, False)
                pltpu.async_copy(z_hbm.at[idxs[kn]], rows.at[nb], semg[nb])

            pltpu.make_async_copy(z_hbm.at[idxs[j]], rows.at[b], semg[b]).wait()
            pltpu.async_copy(rows.at[b], sacc.at[idxd[j]], sems[b], add=True)
        return 0

    lax.fori_loop(0, nwin // 4, quad, 0)

    @pl.when(nwin > 0)
    def _():
        pltpu.make_async_copy(rows.at[0], sacc.at[idxd[2]], sems[0]).wait()
        pltpu.make_async_copy(rows.at[1], sacc.at[idxd[3]], sems[1]).wait()
    plsc.subcore_barrier()

    pltpu.sync_copy(dis_hbm.at[pl.ds(c * PAD + myrow, SLICE)], disv)

    def _tin_copy(bb, t, start):
        a = pltpu.make_async_copy(
            tin_hbm.at[c, pl.ds(myrow + t * XC, XC)], tb.at[bb], semt[bb])
        a.start() if start else a.wait()

    def _st_t(bb, t, start):
        a = pltpu.make_async_copy(
            rows.at[bb, pl.ds(0, XC)],
            tout_hbm.at[c, pl.ds(myrow + t * XC, XC)], semwt[bb])
        a.start() if start else a.wait()

    def _st_z(bb, t, start):
        a = pltpu.make_async_copy(
            rows.at[bb, pl.ds(64, XC)],
            znext_hbm.at[pl.ds(c * PAD + myrow + t * XC, XC)], semwz[bb])
        a.start() if start else a.wait()

    _tin_copy(0, 0, True)
    _tin_copy(1, 1, True)

    def spair(tp, _):
        for bb in range(2):
            t = 2 * tp + bb

            @pl.when(t >= 2)
            def _():
                _st_t(bb, t - 2, False)
                if not last:
                    _st_z(bb, t - 2, False)

            pltpu.sync_copy(sacc.at[pl.ds(myrow + t * XC, XC)], sb)
            _tin_copy(bb, t, False)

            def rbody(lr, _):
                dv = plsc.load_gather(
                    disv, [jnp.full((16,), t * XC + lr, jnp.int32)])
                d2 = dv * dv
                for qq in range(D // 16):
                    sv = sb[lr, pl.ds(qq * 16, 16)]
                    tv = tb[bb, lr, pl.ds(qq * 16, 16)]
                    nt = tv + dv * sv
                    if last:
                        nt = nt * 0.25
                    rows[bb, lr, pl.ds(qq * 16, 16)] = nt
                    if not last:
                        rows[bb, 64 + lr, pl.ds(qq * 16, 16)] = d2 * sv
                return 0
            lax.fori_loop(0, XC, rbody, 0)
            _st_t(bb, t, True)
            if not last:
                _st_z(bb, t, True)

            @pl.when(t + 2 < NCH)
            def _():
                _tin_copy(bb, t + 2, True)
        return 0

    lax.fori_loop(0, NCH // 2, spair, 0)
    _st_t(0, NCH - 2, False)
    _st_t(1, NCH - 1, False)
    if not last:
        _st_z(0, NCH - 2, False)
        _st_z(1, NCH - 1, False)


_params = pltpu.CompilerParams(
    needs_layout_passes=False, use_tc_tiling_on_sc=False)


@functools.lru_cache(maxsize=None)
def _route():
  return pl.kernel(
    _route_body, mesh=_mesh(), compiler_params=_params,
    out_type=(
        jax.ShapeDtypeStruct((32 * CAP,), jnp.int32),
        jax.ShapeDtypeStruct((32 * CAP,), jnp.int32),
        jax.ShapeDtypeStruct((32 * 16,), jnp.int32),
        jax.ShapeDtypeStruct((2 * PAD,), jnp.float32),
        jax.ShapeDtypeStruct((2 * PAD, D), jnp.float32),
    ),
    scratch_types=[
        pltpu.VMEM((2, CHUNK), jnp.int32),
        pltpu.VMEM((2, CHUNK), jnp.int32),
        pltpu.VMEM((CAP,), jnp.int32),
        pltpu.VMEM((CAP,), jnp.int32),
        pltpu.VMEM((W,), jnp.int32),
        pltpu.VMEM((W,), jnp.int32),
        pltpu.VMEM((W,), jnp.float32),
        pltpu.VMEM((SLICE,), jnp.float32),
        pltpu.VMEM((16,), jnp.int32),
        pltpu.VMEM((2, XC, D), jnp.float32),
        pltpu.VMEM_SHARED((PAD,), jnp.float32),
    ] + [pltpu.SemaphoreType.DMA] * 7,
  )


def _layer_scratch():
  return (
    [pltpu.VMEM((W,), jnp.int32)] * 8
    + [
      pltpu.VMEM((2, W, D), jnp.float32),
      pltpu.VMEM((16,), jnp.int32),
      pltpu.VMEM((SLICE,), jnp.float32),
      pltpu.VMEM((XC, D), jnp.float32),
      pltpu.VMEM((2, XC, D), jnp.float32),
      pltpu.VMEM_SHARED((PAD, D), jnp.float32),
    ]
    + [pltpu.SemaphoreType.DMA] * 15
  )


@functools.lru_cache(maxsize=None)
def _layer_mid():
  return pl.kernel(
    functools.partial(_layer_body, last=False), mesh=_mesh(),
    compiler_params=_params,
    out_type=(
        jax.ShapeDtypeStruct((2 * PAD, D), jnp.float32),
        jax.ShapeDtypeStruct((2, PAD, D), jnp.float32),
    ),
    scratch_types=_layer_scratch(),
  )


@functools.lru_cache(maxsize=None)
def _layer_last():
  return pl.kernel(
    functools.partial(_layer_body, last=True), mesh=_mesh(),
    compiler_params=_params,
    out_type=jax.ShapeDtypeStruct((2, PAD, D), jnp.float32),
    scratch_types=_layer_scratch(),
  )


def kernel(user_emb, movie_emb, edge_index):
    ei = edge_index.astype(jnp.int32)
    src = ei[0]
    dst = ei[1]
    x = jnp.concatenate([user_emb, movie_emb], axis=0)
    xpad = jnp.zeros((2, PAD, D), jnp.float32).at[:, :HALF, :].set(
        x.reshape(2, HALF, D))
    esrc, edst, ecnt, dis, z = _route()(src, dst, xpad)
    t = xpad
    z, t = _layer_mid()(z, esrc, edst, ecnt, dis, t)
    z, t = _layer_mid()(z, esrc, edst, ecnt, dis, t)
    t = _layer_last()(z, esrc, edst, ecnt, dis, t)
    return (t[0, :HALF], t[1, :HALF])
